# Initial kernel scaffold; baseline (speedup 1.0000x reference)
import jax
import jax.numpy as jnp
from jax.experimental import pallas as pl


def kernel(y_pred, y_true, qid, indices, num_pos, num_item, ideal_dcg, u_warmup, lambda_q, v_q, s_q):
    raise NotImplementedError("write your pallas kernel here")


# speedup vs baseline: 3.9851x; 3.9851x over previous
import functools

import jax
import jax.numpy as jnp
from jax import lax
from jax.experimental import pallas as pl
from jax.experimental.pallas import tpu as pltpu
from jax.experimental.pallas import tpu_sc as plsc

B = 1024
S = 50
LONGEST = 50
ROW = LONGEST + 2
GAMMA = 0.1
BETA = 0.9
TAU_1 = 0.001
TAU_2 = 0.0001
SIG_ALPHA = 2.0
C_SIG = 2.0
EPS = 1e-10
PAD_Y = -1.0
LN2 = 0.6931471805599453

NC = 2
NS = 16
NW = NC * NS
CHUNK = B * S // NW
NVREG = CHUNK // 16
GCH = 128
NG = -(-CHUNK // GCH)
PADV = NG * GCH
ROWS_W = B // NW


def _sc_gather_body(u_flat, qid_f, idx_f, lam_tab, s_tab,
                    u_out, lam_out, s_out,
                    qid_v, idx_v, off_v, val_v, qb_v, lam_v, sq_v, sem, sem2):
    wid = lax.axis_index("s") * NC + lax.axis_index("c")
    base = wid * CHUNK
    pltpu.sync_copy(qid_f.at[pl.ds(base, CHUNK)], qid_v)
    pltpu.sync_copy(idx_f.at[pl.ds(base, CHUNK)], idx_v)

    iota16 = lax.iota(jnp.int32, 16)
    qb_copies = [
        pltpu.async_copy(qid_f.at[iota16 * S + (base + h * 16 * S)],
                         qb_v.at[pl.ds(h * 16, 16)], sem2)
        for h in range(ROWS_W // 16)
    ]

    def body(v, c):
        q = qid_v[pl.ds(v * 16, 16)]
        x = idx_v[pl.ds(v * 16, 16)]
        off_v[pl.ds(v * 16, 16)] = q * ROW + x + (ROW + 1)
        return c

    lax.fori_loop(0, NVREG, body, 0)
    zero16 = jnp.zeros((16,), jnp.int32)
    for t in range(NVREG, PADV // 16):
        off_v[pl.ds(t * 16, 16)] = zero16

    copies = [
        pltpu.async_copy(u_flat.at[off_v.at[pl.ds(j * GCH, GCH)]],
                         val_v.at[pl.ds(j * GCH, GCH)], sem)
        for j in range(NG)
    ]

    for c in qb_copies:
        c.wait()
    for h in range(ROWS_W // 16):
        off16 = qb_v[pl.ds(h * 16, 16)] + 1
        copies.append(pltpu.async_copy(lam_tab.at[off16],
                                       lam_v.at[pl.ds(h * 16, 16)], sem))
        copies.append(pltpu.async_copy(s_tab.at[off16],
                                       sq_v.at[pl.ds(h * 16, 16)], sem))
    for c in copies:
        c.wait()

    pltpu.sync_copy(val_v.at[pl.ds(0, CHUNK)], u_out.at[pl.ds(base, CHUNK)])
    pltpu.sync_copy(lam_v, lam_out.at[pl.ds(wid * ROWS_W, ROWS_W)])
    pltpu.sync_copy(sq_v, s_out.at[pl.ds(wid * ROWS_W, ROWS_W)])


def _make_sc_gather():
    return pl.kernel(
        _sc_gather_body,
        out_type=[
            jax.ShapeDtypeStruct((B * S,), jnp.float32),
            jax.ShapeDtypeStruct((B,), jnp.float32),
            jax.ShapeDtypeStruct((B,), jnp.float32),
        ],
        mesh=plsc.VectorSubcoreMesh(core_axis_name="c", subcore_axis_name="s"),
        scratch_types=[
            pltpu.VMEM((CHUNK,), jnp.int32),
            pltpu.VMEM((CHUNK,), jnp.int32),
            pltpu.VMEM((PADV,), jnp.int32),
            pltpu.VMEM((PADV,), jnp.float32),
            pltpu.VMEM((ROWS_W,), jnp.int32),
            pltpu.VMEM((ROWS_W,), jnp.float32),
            pltpu.VMEM((ROWS_W,), jnp.float32),
            pltpu.SemaphoreType.DMA,
            pltpu.SemaphoreType.DMA,
        ],
    )


def _sig(x):
    ex = jnp.exp(-jnp.abs(x))
    return jnp.where(x >= 0, 1.0 / (1.0 + ex), ex / (1.0 + ex))


def _tc_body(yp_ref, yt_ref, old_ref, lam_ref, sq_ref, np_ref, ni_ref,
             dcg_ref, out_ref, acc_ref):
    i = pl.program_id(0)
    yp = yp_ref[...]
    yt = yt_ref[...]
    m = yt != PAD_Y
    mf = m.astype(jnp.float32)
    cnt = jnp.sum(mf, axis=1, keepdims=True)
    ypi = yp[:, :, None]
    ypj = yp[:, None, :]
    h = jnp.maximum(ypj - ypi + 1.0, 0.0)
    w = mf[:, None, :] * mf[:, :, None]
    g = jnp.sum(w * h * h, axis=2) / cnt + EPS
    gu = (1.0 - GAMMA) * old_ref[...] + GAMMA * g
    G = jnp.where(m, jnp.exp2(jnp.maximum(yt, 0.0)) - 1.0, 0.0)
    nif = ni_ref[...].astype(jnp.float32)
    Dn = 2.0 + nif * gu
    l2d = jnp.log(Dn) * (1.0 / LN2)
    nab = G * nif / (l2d * l2d * Dn * LN2)
    pld = jnp.where(m, yp - lam_ref[...], 0.0)
    sA = _sig(pld * SIG_ALPHA)
    nab = nab * (C_SIG * sA)
    w1 = C_SIG * sA * (1.0 - sA)
    st = _sig(pld * (1.0 / TAU_1))
    temp = st * (1.0 - st) * (1.0 / TAU_1)
    L_h = TAU_2 + jnp.sum(mf * temp, axis=1, keepdims=True) / cnt
    s_used = BETA * L_h + (1.0 - BETA) * sq_ref[...]
    ypz = jnp.where(m, yp, 0.0)
    hess = jnp.sum(mf * temp * ypz, axis=1, keepdims=True) / cnt / s_used
    fgu = -G / l2d
    inner = jnp.sum(nab * g + w1 * fgu * (ypz - hess), axis=1,
                    keepdims=True) * (1.0 / S)
    sa = jnp.sum(np_ref[...].astype(jnp.float32) / (dcg_ref[...] + EPS))
    si = jnp.sum(inner)

    @pl.when(i == 0)
    def _():
        acc_ref[0] = 0.0
        acc_ref[1] = 0.0

    acc_ref[0] += sa
    acc_ref[1] += si

    @pl.when(i == pl.num_programs(0) - 1)
    def _():
        out_ref[...] = jnp.full((1, 1), (acc_ref[0] * (1.0 / B)) *
                                (acc_ref[1] * (1.0 / B)), jnp.float32)


_RB = 128


def _tc_loss(y_pred, y_true, old_u, lam_g, s_g, num_pos, num_item, ideal_dcg):
    return pl.pallas_call(
        _tc_body,
        grid=(B // _RB,),
        in_specs=[
            pl.BlockSpec((_RB, S), lambda i: (i, 0)),
            pl.BlockSpec((_RB, S), lambda i: (i, 0)),
            pl.BlockSpec((_RB, S), lambda i: (i, 0)),
            pl.BlockSpec((_RB, 1), lambda i: (i, 0)),
            pl.BlockSpec((_RB, 1), lambda i: (i, 0)),
            pl.BlockSpec((_RB, 1), lambda i: (i, 0)),
            pl.BlockSpec((_RB, 1), lambda i: (i, 0)),
            pl.BlockSpec((_RB, 1), lambda i: (i, 0)),
        ],
        out_specs=pl.BlockSpec((1, 1), lambda i: (0, 0)),
        out_shape=jax.ShapeDtypeStruct((1, 1), jnp.float32),
        scratch_shapes=[pltpu.SMEM((2,), jnp.float32)],
    )(y_pred, y_true, old_u,
      lam_g.reshape(B, 1), s_g.reshape(B, 1),
      num_pos.reshape(B, 1), num_item.reshape(B, 1),
      ideal_dcg.reshape(B, 1))


def kernel(y_pred, y_true, qid, indices, num_pos, num_item, ideal_dcg,
           u_warmup, lambda_q, v_q, s_q):
    del v_q
    old_u, lam_g, s_g = _make_sc_gather()(
        u_warmup.reshape(-1), qid.reshape(-1), indices.reshape(-1),
        lambda_q, s_q)
    loss = _tc_loss(y_pred, y_true, old_u.reshape(B, S), lam_g, s_g,
                    num_pos, num_item, ideal_dcg)
    return loss[0, 0]


# speedup vs baseline: 5.2057x; 1.3063x over previous
import functools

import jax
import jax.numpy as jnp
from jax import lax
from jax.experimental import pallas as pl
from jax.experimental.pallas import tpu as pltpu
from jax.experimental.pallas import tpu_sc as plsc

B = 1024
S = 50
LONGEST = 50
ROW = LONGEST + 2
GAMMA = 0.1
BETA = 0.9
TAU_1 = 0.001
TAU_2 = 0.0001
SIG_ALPHA = 2.0
C_SIG = 2.0
EPS = 1e-10
PAD_Y = -1.0
LN2 = 0.6931471805599453

NC = 2
NS = 16
NW = NC * NS
CHUNK = B * S // NW
NVREG = CHUNK // 16
GCH = 128
NG = -(-CHUNK // GCH)
PADV = NG * GCH
ROWS_W = B // NW


def _sc_gather_body(u_flat, qid_f, idx_f, lam_tab, s_tab,
                    u_out, lam_out, s_out,
                    qid_v, idx_v, off_v, val_v, qb_v, lam_v, sq_v, sem, sem2):
    wid = lax.axis_index("s") * NC + lax.axis_index("c")
    base = wid * CHUNK
    pltpu.sync_copy(qid_f.at[pl.ds(base, CHUNK)], qid_v)
    pltpu.sync_copy(idx_f.at[pl.ds(base, CHUNK)], idx_v)

    iota16 = lax.iota(jnp.int32, 16)
    qb_copies = [
        pltpu.async_copy(qid_f.at[iota16 * S + (base + h * 16 * S)],
                         qb_v.at[pl.ds(h * 16, 16)], sem2)
        for h in range(ROWS_W // 16)
    ]

    def body(v, c):
        q = qid_v[pl.ds(v * 16, 16)]
        x = idx_v[pl.ds(v * 16, 16)]
        off_v[pl.ds(v * 16, 16)] = q * ROW + x + 1
        return c

    lax.fori_loop(0, NVREG, body, 0)
    zero16 = jnp.zeros((16,), jnp.int32)
    for t in range(NVREG, PADV // 16):
        off_v[pl.ds(t * 16, 16)] = zero16

    copies = [
        pltpu.async_copy(u_flat.at[off_v.at[pl.ds(j * GCH, GCH)]],
                         val_v.at[pl.ds(j * GCH, GCH)], sem)
        for j in range(NG)
    ]

    for c in qb_copies:
        c.wait()
    for h in range(ROWS_W // 16):
        off16 = qb_v[pl.ds(h * 16, 16)] + 1
        copies.append(pltpu.async_copy(lam_tab.at[off16],
                                       lam_v.at[pl.ds(h * 16, 16)], sem))
        copies.append(pltpu.async_copy(s_tab.at[off16],
                                       sq_v.at[pl.ds(h * 16, 16)], sem))
    for c in copies:
        c.wait()

    pltpu.sync_copy(val_v.at[pl.ds(0, CHUNK)], u_out.at[pl.ds(base, CHUNK)])
    pltpu.sync_copy(lam_v, lam_out.at[pl.ds(wid * ROWS_W, ROWS_W)])
    pltpu.sync_copy(sq_v, s_out.at[pl.ds(wid * ROWS_W, ROWS_W)])


def _make_sc_gather():
    return pl.kernel(
        _sc_gather_body,
        out_type=[
            jax.ShapeDtypeStruct((B * S,), jnp.float32),
            jax.ShapeDtypeStruct((B,), jnp.float32),
            jax.ShapeDtypeStruct((B,), jnp.float32),
        ],
        mesh=plsc.VectorSubcoreMesh(core_axis_name="c", subcore_axis_name="s"),
        scratch_types=[
            pltpu.VMEM((CHUNK,), jnp.int32),
            pltpu.VMEM((CHUNK,), jnp.int32),
            pltpu.VMEM((PADV,), jnp.int32),
            pltpu.VMEM((PADV,), jnp.float32),
            pltpu.VMEM((ROWS_W,), jnp.int32),
            pltpu.VMEM((ROWS_W,), jnp.float32),
            pltpu.VMEM((ROWS_W,), jnp.float32),
            pltpu.SemaphoreType.DMA,
            pltpu.SemaphoreType.DMA,
        ],
    )


def _sig(x):
    ex = jnp.exp(-jnp.abs(x))
    return jnp.where(x >= 0, 1.0 / (1.0 + ex), ex / (1.0 + ex))


def _tc_body(yp_ref, yt_ref, old_ref, lam_ref, sq_ref, np_ref, ni_ref,
             dcg_ref, out_ref, acc_ref):
    i = pl.program_id(0)
    yp = yp_ref[...]
    yt = yt_ref[...]
    m = yt != PAD_Y
    mf = m.astype(jnp.float32)
    cnt = jnp.sum(mf, axis=1, keepdims=True)
    ypi = yp[:, :, None]
    ypj = yp[:, None, :]
    h = jnp.maximum(ypj - ypi + 1.0, 0.0)
    w = mf[:, None, :] * mf[:, :, None]
    g = jnp.sum(w * h * h, axis=2) / cnt + EPS
    gu = (1.0 - GAMMA) * old_ref[...] + GAMMA * g
    G = jnp.where(m, jnp.exp2(jnp.maximum(yt, 0.0)) - 1.0, 0.0)
    nif = ni_ref[...].astype(jnp.float32)
    Dn = 2.0 + nif * gu
    l2d = jnp.log(Dn) * (1.0 / LN2)
    nab = G * nif / (l2d * l2d * Dn * LN2)
    pld = jnp.where(m, yp - lam_ref[...], 0.0)
    sA = _sig(pld * SIG_ALPHA)
    nab = nab * (C_SIG * sA)
    w1 = C_SIG * sA * (1.0 - sA)
    st = _sig(pld * (1.0 / TAU_1))
    temp = st * (1.0 - st) * (1.0 / TAU_1)
    L_h = TAU_2 + jnp.sum(mf * temp, axis=1, keepdims=True) / cnt
    s_used = BETA * L_h + (1.0 - BETA) * sq_ref[...]
    ypz = jnp.where(m, yp, 0.0)
    hess = jnp.sum(mf * temp * ypz, axis=1, keepdims=True) / cnt / s_used
    fgu = -G / l2d
    inner = jnp.sum(nab * g + w1 * fgu * (ypz - hess), axis=1,
                    keepdims=True) * (1.0 / S)
    sa = jnp.sum(np_ref[...].astype(jnp.float32) / (dcg_ref[...] + EPS))
    si = jnp.sum(inner)

    @pl.when(i == 0)
    def _():
        acc_ref[0] = 0.0
        acc_ref[1] = 0.0

    acc_ref[0] += sa
    acc_ref[1] += si

    @pl.when(i == pl.num_programs(0) - 1)
    def _():
        out_ref[...] = jnp.full((1, 1), (acc_ref[0] * (1.0 / B)) *
                                (acc_ref[1] * (1.0 / B)), jnp.float32)


_RB = 128


def _tc_loss(y_pred, y_true, old_u, lam_g, s_g, num_pos, num_item, ideal_dcg):
    return pl.pallas_call(
        _tc_body,
        grid=(B // _RB,),
        in_specs=[
            pl.BlockSpec((_RB, S), lambda i: (i, 0)),
            pl.BlockSpec((_RB, S), lambda i: (i, 0)),
            pl.BlockSpec((_RB, S), lambda i: (i, 0)),
            pl.BlockSpec((_RB, 1), lambda i: (i, 0)),
            pl.BlockSpec((_RB, 1), lambda i: (i, 0)),
            pl.BlockSpec((_RB, 1), lambda i: (i, 0)),
            pl.BlockSpec((_RB, 1), lambda i: (i, 0)),
            pl.BlockSpec((_RB, 1), lambda i: (i, 0)),
        ],
        out_specs=pl.BlockSpec((1, 1), lambda i: (0, 0)),
        out_shape=jax.ShapeDtypeStruct((1, 1), jnp.float32),
        scratch_shapes=[pltpu.SMEM((2,), jnp.float32)],
    )(y_pred, y_true, old_u,
      lam_g.reshape(B, 1), s_g.reshape(B, 1),
      num_pos.reshape(B, 1), num_item.reshape(B, 1),
      ideal_dcg.reshape(B, 1))


def kernel(y_pred, y_true, qid, indices, num_pos, num_item, ideal_dcg,
           u_warmup, lambda_q, v_q, s_q):
    del v_q
    old_u, lam_g, s_g = _make_sc_gather()(
        u_warmup[1:B * S + 1].reshape(-1), qid.reshape(-1),
        indices.reshape(-1), lambda_q, s_q)
    loss = _tc_loss(y_pred, y_true, old_u.reshape(B, S), lam_g, s_g,
                    num_pos, num_item, ideal_dcg)
    return loss[0, 0]
